# baseline (speedup 1.0000x reference)
import functools

import jax
import jax.numpy as jnp
from jax.experimental import pallas as pl
from jax.experimental.pallas import tpu as pltpu


def _leaky(x, slope=0.01):
    return jnp.where(x >= 0, x, slope * x)


def _moe_body(x_ref, rW1_ref, rb1_ref, rW2_ref, rb2_ref, eW_ref, eb_ref,
              out_ref, *, n_exp):
    xb = x_ref[...]
    h = jnp.dot(xb, rW1_ref[...], preferred_element_type=jnp.float32)
    h = _leaky(h + rb1_ref[...])
    logits = jnp.dot(h, rW2_ref[...], preferred_element_type=jnp.float32)
    logits = logits + rb2_ref[...]

    bm = logits.shape[0]
    ids = jax.lax.broadcasted_iota(jnp.int32, (bm, n_exp), 1)
    m1 = jnp.max(logits, axis=1, keepdims=True)
    i1 = jnp.min(jnp.where(logits == m1, ids, n_exp), axis=1, keepdims=True)
    masked = jnp.where(ids == i1, -jnp.inf, logits)
    m2 = jnp.max(masked, axis=1, keepdims=True)
    i2 = jnp.min(jnp.where(masked == m2, ids, n_exp), axis=1, keepdims=True)
    e2 = jnp.exp(m2 - m1)
    p1 = 1.0 / (1.0 + e2)
    p2 = e2 / (1.0 + e2)
    coef = jnp.where(ids == i1, p1, 0.0) + jnp.where(ids == i2, p2, 0.0)

    xbf = xb.astype(jnp.bfloat16)
    acc0 = jnp.dot(coef, eb_ref[...], preferred_element_type=jnp.float32)
    acc1 = coef[:, 0:1] * jnp.dot(xbf, eW_ref[0],
                                  preferred_element_type=jnp.float32)
    for e in range(1, n_exp):
        y = jnp.dot(xbf, eW_ref[e], preferred_element_type=jnp.float32)
        if e % 2 == 0:
            acc0 = acc0 + coef[:, e:e + 1] * y
        else:
            acc1 = acc1 + coef[:, e:e + 1] * y
    out_ref[...] = _leaky(acc0 + acc1)


@jax.jit
def kernel(x, rW1, rb1, rW2, rb2, eW, eb):
    n, d_in = x.shape
    h_dim = rW1.shape[1]
    n_exp = eW.shape[0]
    d_out = eW.shape[2]
    bm = min(1024, n)
    grid = (n // bm,)

    eW_bf = eW.astype(jnp.bfloat16)

    out = pl.pallas_call(
        functools.partial(_moe_body, n_exp=n_exp),
        grid=grid,
        in_specs=[
            pl.BlockSpec((bm, d_in), lambda i: (i, 0)),
            pl.BlockSpec((d_in, h_dim), lambda i: (0, 0)),
            pl.BlockSpec((1, h_dim), lambda i: (0, 0)),
            pl.BlockSpec((h_dim, n_exp), lambda i: (0, 0)),
            pl.BlockSpec((1, n_exp), lambda i: (0, 0)),
            pl.BlockSpec((n_exp, d_in, d_out), lambda i: (0, 0, 0)),
            pl.BlockSpec((n_exp, d_out), lambda i: (0, 0)),
        ],
        out_specs=pl.BlockSpec((bm, d_out), lambda i: (i, 0)),
        out_shape=jax.ShapeDtypeStruct((n, d_out), jnp.float32),
        compiler_params=pltpu.CompilerParams(
            dimension_semantics=("arbitrary",)),
    )(x, rW1, rb1.reshape(1, h_dim), rW2, rb2.reshape(1, n_exp), eW_bf, eb)
    return out


# speedup vs baseline: 1.0598x; 1.0598x over previous
import functools

import jax
import jax.numpy as jnp
from jax.experimental import pallas as pl
from jax.experimental.pallas import tpu as pltpu


def _leaky(x, slope=0.01):
    return jnp.where(x >= 0, x, slope * x)


def _moe_body(x_ref, rW1_ref, rb1_ref, rW2_ref, rb2_ref, eW_ref, eb_ref,
              out_ref, *, n_exp):
    xb = x_ref[...]
    h = jnp.dot(xb, rW1_ref[...], preferred_element_type=jnp.float32)
    h = _leaky(h + rb1_ref[...])
    logits = jnp.dot(h, rW2_ref[...], preferred_element_type=jnp.float32)
    logits = logits + rb2_ref[...]

    bm = logits.shape[0]
    ids = jax.lax.broadcasted_iota(jnp.int32, (bm, n_exp), 1)
    m1 = jnp.max(logits, axis=1, keepdims=True)
    i1 = jnp.min(jnp.where(logits == m1, ids, n_exp), axis=1, keepdims=True)
    masked = jnp.where(ids == i1, -jnp.inf, logits)
    m2 = jnp.max(masked, axis=1, keepdims=True)
    i2 = jnp.min(jnp.where(masked == m2, ids, n_exp), axis=1, keepdims=True)
    e2 = jnp.exp(m2 - m1)
    p1 = 1.0 / (1.0 + e2)
    p2 = e2 / (1.0 + e2)
    coef = jnp.where(ids == i1, p1, 0.0) + jnp.where(ids == i2, p2, 0.0)

    acc0 = jnp.dot(coef, eb_ref[...], preferred_element_type=jnp.float32)
    acc1 = coef[:, 0:1] * jnp.dot(xb, eW_ref[0],
                                  preferred_element_type=jnp.float32)
    for e in range(1, n_exp):
        y = jnp.dot(xb, eW_ref[e], preferred_element_type=jnp.float32)
        if e % 2 == 0:
            acc0 = acc0 + coef[:, e:e + 1] * y
        else:
            acc1 = acc1 + coef[:, e:e + 1] * y
    out_ref[...] = _leaky(acc0 + acc1)


@jax.jit
def kernel(x, rW1, rb1, rW2, rb2, eW, eb):
    n, d_in = x.shape
    h_dim = rW1.shape[1]
    n_exp = eW.shape[0]
    d_out = eW.shape[2]
    bm = min(256, n)
    grid = (n // bm,)

    out = pl.pallas_call(
        functools.partial(_moe_body, n_exp=n_exp),
        grid=grid,
        in_specs=[
            pl.BlockSpec((bm, d_in), lambda i: (i, 0)),
            pl.BlockSpec((d_in, h_dim), lambda i: (0, 0)),
            pl.BlockSpec((1, h_dim), lambda i: (0, 0)),
            pl.BlockSpec((h_dim, n_exp), lambda i: (0, 0)),
            pl.BlockSpec((1, n_exp), lambda i: (0, 0)),
            pl.BlockSpec((n_exp, d_in, d_out), lambda i: (0, 0, 0)),
            pl.BlockSpec((n_exp, d_out), lambda i: (0, 0)),
        ],
        out_specs=pl.BlockSpec((bm, d_out), lambda i: (i, 0)),
        out_shape=jax.ShapeDtypeStruct((n, d_out), jnp.float32),
        compiler_params=pltpu.CompilerParams(
            dimension_semantics=("arbitrary",)),
    )(x, rW1, rb1.reshape(1, h_dim), rW2, rb2.reshape(1, n_exp), eW, eb)
    return out


# speedup vs baseline: 1.1108x; 1.0481x over previous
import functools

import jax
import jax.numpy as jnp
from jax.experimental import pallas as pl
from jax.experimental.pallas import tpu as pltpu


def _leaky(x, slope=0.01):
    return jnp.where(x >= 0, x, slope * x)


def _moe_body(x_ref, rW1_ref, rb1_ref, rW2_ref, rb2_ref, eW_ref, eb_ref,
              out_ref, *, n_exp):
    xb = x_ref[...]
    h = jnp.dot(xb, rW1_ref[...], preferred_element_type=jnp.float32)
    h = _leaky(h + rb1_ref[...])
    logits = jnp.dot(h, rW2_ref[...], preferred_element_type=jnp.float32)
    logits = logits + rb2_ref[...]

    bm = logits.shape[0]
    ids = jax.lax.broadcasted_iota(jnp.int32, (bm, n_exp), 1)
    m1 = jnp.max(logits, axis=1, keepdims=True)
    i1 = jnp.min(jnp.where(logits == m1, ids, n_exp), axis=1, keepdims=True)
    masked = jnp.where(ids == i1, -jnp.inf, logits)
    m2 = jnp.max(masked, axis=1, keepdims=True)
    i2 = jnp.min(jnp.where(masked == m2, ids, n_exp), axis=1, keepdims=True)
    e2 = jnp.exp(m2 - m1)
    p1 = 1.0 / (1.0 + e2)
    p2 = e2 / (1.0 + e2)
    coef = jnp.where(ids == i1, p1, 0.0) + jnp.where(ids == i2, p2, 0.0)

    acc0 = jnp.dot(coef, eb_ref[...], preferred_element_type=jnp.float32)
    acc1 = coef[:, 0:1] * jnp.dot(xb, eW_ref[0],
                                  preferred_element_type=jnp.float32)
    for e in range(1, n_exp):
        y = jnp.dot(xb, eW_ref[e], preferred_element_type=jnp.float32)
        if e % 2 == 0:
            acc0 = acc0 + coef[:, e:e + 1] * y
        else:
            acc1 = acc1 + coef[:, e:e + 1] * y
    out_ref[...] = _leaky(acc0 + acc1)


@jax.jit
def kernel(x, rW1, rb1, rW2, rb2, eW, eb):
    n, d_in = x.shape
    h_dim = rW1.shape[1]
    n_exp = eW.shape[0]
    d_out = eW.shape[2]
    bm = min(512, n)
    grid = (n // bm,)

    out = pl.pallas_call(
        functools.partial(_moe_body, n_exp=n_exp),
        grid=grid,
        in_specs=[
            pl.BlockSpec((bm, d_in), lambda i: (i, 0)),
            pl.BlockSpec((d_in, h_dim), lambda i: (0, 0)),
            pl.BlockSpec((1, h_dim), lambda i: (0, 0)),
            pl.BlockSpec((h_dim, n_exp), lambda i: (0, 0)),
            pl.BlockSpec((1, n_exp), lambda i: (0, 0)),
            pl.BlockSpec((n_exp, d_in, d_out), lambda i: (0, 0, 0)),
            pl.BlockSpec((n_exp, d_out), lambda i: (0, 0)),
        ],
        out_specs=pl.BlockSpec((bm, d_out), lambda i: (i, 0)),
        out_shape=jax.ShapeDtypeStruct((n, d_out), jnp.float32),
        compiler_params=pltpu.CompilerParams(
            dimension_semantics=("arbitrary",)),
    )(x, rW1, rb1.reshape(1, h_dim), rW2, rb2.reshape(1, n_exp), eW, eb)
    return out
